# baseline (speedup 1.0000x reference)
import functools

import jax
import jax.numpy as jnp
from jax import lax
from jax.experimental import pallas as pl
from jax.experimental.pallas import tpu as pltpu
from jax.experimental.pallas import tpu_sc as plsc

S, D, E, K, F = 2048, 768, 8, 2, 2048
P = S * K
BLK = 128
NT = (P // BLK) + E - 1
TE_N = 64
ROWS = NT * BLK
NW = 32
CHUNK = P // NW
CB = 512


def _router_body(x_ref, rs_ref, wr_ref, cw_ref, pos_ref, te_ref):
    x = x_ref[...]
    xr = x * lax.rsqrt(jnp.mean(x * x, axis=-1, keepdims=True) + 1e-6)
    xr = xr * (lax.rsqrt(jnp.float32(D)) * rs_ref[...])
    logits = jnp.dot(xr, wr_ref[...], preferred_element_type=jnp.float32)
    m = jnp.max(logits, axis=-1, keepdims=True)
    ex = jnp.exp(logits - m)
    probs = ex / jnp.sum(ex, axis=-1, keepdims=True)
    idx = lax.broadcasted_iota(jnp.int32, (S, E), 1)
    a1 = jnp.min(jnp.where(logits == m, idx, E), axis=-1, keepdims=True)
    l2 = jnp.where(idx == a1, -jnp.inf, logits)
    m2 = jnp.max(l2, axis=-1, keepdims=True)
    a2 = jnp.min(jnp.where(l2 == m2, idx, E), axis=-1, keepdims=True)
    oh1 = (idx == a1).astype(jnp.float32)
    oh2 = (idx == a2).astype(jnp.float32)
    p1 = jnp.sum(probs * oh1, axis=-1, keepdims=True)
    p2 = jnp.sum(probs * oh2, axis=-1, keepdims=True)
    rf = p1 + p2
    cw_ref[...] = jnp.concatenate(
        [jnp.broadcast_to(p1 / rf, (S, 128)), jnp.broadcast_to(p2 / rf, (S, 128))],
        axis=0)
    ohsum = oh1 + oh2
    r0 = lax.broadcasted_iota(jnp.int32, (CB, CB), 0)
    c0 = lax.broadcasted_iota(jnp.int32, (CB, CB), 1)
    tri = (c0 < r0).astype(jnp.float32)
    carry = jnp.zeros((1, E), jnp.float32)
    excl_rows = []
    for b in range(S // CB):
        blk = ohsum[b * CB:(b + 1) * CB, :]
        excl_rows.append(jnp.dot(tri, blk, preferred_element_type=jnp.float32) + carry)
        carry = carry + jnp.sum(blk, axis=0, keepdims=True)
    excl_cum = jnp.concatenate(excl_rows, axis=0)
    counts = carry
    cap = jnp.floor((counts + (BLK - 1)) * (1.0 / BLK)) * BLK
    iu = lax.broadcasted_iota(jnp.int32, (E, E), 0)
    ju = lax.broadcasted_iota(jnp.int32, (E, E), 1)
    upp = (iu < ju).astype(jnp.float32)
    starts = jnp.dot(cap, upp, preferred_element_type=jnp.float32)
    rank1 = jnp.sum(excl_cum * oh1, axis=-1, keepdims=True)
    rank2 = jnp.sum(excl_cum * oh2, axis=-1, keepdims=True)
    s1 = jnp.sum(starts * oh1, axis=-1, keepdims=True)
    s2 = jnp.sum(starts * oh2, axis=-1, keepdims=True)
    pos_ref[...] = jnp.concatenate([s1 + rank1, s2 + rank2], axis=0).astype(jnp.int32)
    ends = starts + cap
    jt = (lax.broadcasted_iota(jnp.int32, (TE_N, E), 0) * BLK).astype(jnp.float32)
    te = jnp.sum((jt >= ends).astype(jnp.int32), axis=-1, keepdims=True)
    te = jnp.minimum(te, E - 1)
    row_id = lax.broadcasted_iota(jnp.int32, (TE_N, 1), 0)
    used = (ends[0, E - 1] * (1.0 / BLK)).astype(jnp.int32)
    te_ref[...] = jnp.where(row_id == TE_N - 1, used, te)


def _ffn_body(te_ref, x_ref, wg_ref, wl_ref, ps_ref, cw_ref, y_ref):
    i = pl.program_id(0)

    @pl.when(i < te_ref[TE_N - 1])
    def _():
        e = te_ref[i]
        xb = x_ref[...]
        wg = wg_ref[0]
        h1 = lax.dot_general(xb, wg[:F], (((1,), (1,)), ((), ())),
                             preferred_element_type=jnp.float32)
        h2 = lax.dot_general(xb, wg[F:], (((1,), (1,)), ((), ())),
                             preferred_element_type=jnp.float32)
        act = jax.nn.gelu(h1) * h2
        y = lax.dot_general(act, wl_ref[0], (((1,), (0,)), ((), ())),
                            preferred_element_type=jnp.float32)
        y_ref[...] = y * (cw_ref[:, 0:1] * ps_ref[e])


NTOK = S // NW


def _dispatch_sc_body(x_hbm, pos_hbm, cwe_hbm, xs_hbm, cws_hbm,
                      xrow_v, p0_v, p1_v, cw0_v, cw1_v, sem):
    wid = lax.axis_index("s") * 2 + lax.axis_index("c")
    tb = wid * NTOK
    l1 = pltpu.async_copy(x_hbm.at[pl.ds(tb, NTOK)], xrow_v, sem)
    l2 = pltpu.async_copy(pos_hbm.at[pl.ds(tb, NTOK)], p0_v, sem)
    l3 = pltpu.async_copy(pos_hbm.at[pl.ds(S + tb, NTOK)], p1_v, sem)
    l4 = pltpu.async_copy(cwe_hbm.at[pl.ds(tb, NTOK)], cw0_v, sem)
    l5 = pltpu.async_copy(cwe_hbm.at[pl.ds(S + tb, NTOK)], cw1_v, sem)
    l1.wait()
    l2.wait()
    l3.wait()
    l4.wait()
    l5.wait()
    c1 = pltpu.async_copy(xrow_v, xs_hbm.at[p0_v], sem)
    c2 = pltpu.async_copy(xrow_v, xs_hbm.at[p1_v], sem)
    c3 = pltpu.async_copy(cw0_v, cws_hbm.at[p0_v], sem)
    c4 = pltpu.async_copy(cw1_v, cws_hbm.at[p1_v], sem)
    c1.wait()
    c2.wait()
    c3.wait()
    c4.wait()


def _collect_sc_body(y_hbm, pos_hbm, o_hbm, p0_v, p1_v, rows_a, rows_b, sem):
    wid = lax.axis_index("s") * 2 + lax.axis_index("c")
    tb = wid * NTOK
    pltpu.sync_copy(pos_hbm.at[pl.ds(tb, NTOK)], p0_v)
    pltpu.sync_copy(pos_hbm.at[pl.ds(S + tb, NTOK)], p1_v)
    g1 = pltpu.async_copy(y_hbm.at[p0_v], rows_a, sem)
    g2 = pltpu.async_copy(y_hbm.at[p1_v], rows_b, sem)
    g1.wait()
    g2.wait()

    @plsc.parallel_loop(0, NTOK)
    def _(t):
        for c in range(D // 16):
            a = rows_a[t, pl.ds(c * 16, 16)]
            b = rows_b[t, pl.ds(c * 16, 16)]
            rows_a[t, pl.ds(c * 16, 16)] = a + b

    pltpu.sync_copy(rows_a, o_hbm.at[pl.ds(tb, NTOK)])


def _sc_mesh():
    return plsc.VectorSubcoreMesh(core_axis_name="c", subcore_axis_name="s")


def kernel(x, router_scale, per_expert_scale, w_router, w_gating, w_linear):
    x2 = x.reshape(S, D)
    cw, pos, te = pl.pallas_call(
        _router_body,
        out_shape=[
            jax.ShapeDtypeStruct((P, 128), jnp.float32),
            jax.ShapeDtypeStruct((P, 1), jnp.int32),
            jax.ShapeDtypeStruct((TE_N, 1), jnp.int32),
        ],
    )(x2, router_scale.reshape(1, D), w_router)
    pos_flat = pos.reshape(P)

    dispatch = pl.kernel(
        _dispatch_sc_body,
        out_type=(
            jax.ShapeDtypeStruct((ROWS, D), jnp.float32),
            jax.ShapeDtypeStruct((ROWS, 128), jnp.float32),
        ),
        mesh=_sc_mesh(),
        scratch_types=[
            pltpu.VMEM((NTOK, D), jnp.float32),
            pltpu.VMEM((NTOK,), jnp.int32),
            pltpu.VMEM((NTOK,), jnp.int32),
            pltpu.VMEM((NTOK, 128), jnp.float32),
            pltpu.VMEM((NTOK, 128), jnp.float32),
            pltpu.SemaphoreType.DMA,
        ],
    )
    xs, cws = dispatch(x2, pos_flat, cw)

    te_arr = te.reshape(TE_N)
    grid_spec = pltpu.PrefetchScalarGridSpec(
        num_scalar_prefetch=1,
        grid=(NT,),
        in_specs=[
            pl.BlockSpec((BLK, D), lambda i, te: (i, 0)),
            pl.BlockSpec((1, 2 * F, D), lambda i, te: (te[i], 0, 0)),
            pl.BlockSpec((1, F, D), lambda i, te: (te[i], 0, 0)),
            pl.BlockSpec(memory_space=pltpu.SMEM),
            pl.BlockSpec((BLK, 128), lambda i, te: (i, 0)),
        ],
        out_specs=pl.BlockSpec((BLK, D), lambda i, te: (i, 0)),
    )
    y = pl.pallas_call(
        _ffn_body,
        grid_spec=grid_spec,
        out_shape=jax.ShapeDtypeStruct((ROWS, D), jnp.float32),
    )(te_arr, xs, w_gating.reshape(E, 2 * F, D), w_linear, per_expert_scale, cws)

    collect = pl.kernel(
        _collect_sc_body,
        out_type=jax.ShapeDtypeStruct((S, D), jnp.float32),
        mesh=_sc_mesh(),
        scratch_types=[
            pltpu.VMEM((NTOK,), jnp.int32),
            pltpu.VMEM((NTOK,), jnp.int32),
            pltpu.VMEM((NTOK, D), jnp.float32),
            pltpu.VMEM((NTOK, D), jnp.float32),
            pltpu.SemaphoreType.DMA,
        ],
    )
    out = collect(y, pos_flat)
    return out.reshape(1, S, D)


# speedup vs baseline: 1.2666x; 1.2666x over previous
import functools

import jax
import jax.numpy as jnp
from jax import lax
from jax.experimental import pallas as pl
from jax.experimental.pallas import tpu as pltpu
from jax.experimental.pallas import tpu_sc as plsc

S, D, E, K, F = 2048, 768, 8, 2, 2048
P = S * K
BLK = 256
NT = (P // BLK) + E - 1
TE_N = 64
ROWS = NT * BLK
NW = 32
CHUNK = P // NW
CB = 512


def _router_body(x_ref, rs_ref, wr_ref, cw_ref, pos_ref, te_ref):
    x = x_ref[...]
    xr = x * lax.rsqrt(jnp.mean(x * x, axis=-1, keepdims=True) + 1e-6)
    xr = xr * (lax.rsqrt(jnp.float32(D)) * rs_ref[...])
    logits = jnp.dot(xr, wr_ref[...], preferred_element_type=jnp.float32)
    m = jnp.max(logits, axis=-1, keepdims=True)
    ex = jnp.exp(logits - m)
    probs = ex / jnp.sum(ex, axis=-1, keepdims=True)
    idx = lax.broadcasted_iota(jnp.int32, (S, E), 1)
    a1 = jnp.min(jnp.where(logits == m, idx, E), axis=-1, keepdims=True)
    l2 = jnp.where(idx == a1, -jnp.inf, logits)
    m2 = jnp.max(l2, axis=-1, keepdims=True)
    a2 = jnp.min(jnp.where(l2 == m2, idx, E), axis=-1, keepdims=True)
    oh1 = (idx == a1).astype(jnp.float32)
    oh2 = (idx == a2).astype(jnp.float32)
    p1 = jnp.sum(probs * oh1, axis=-1, keepdims=True)
    p2 = jnp.sum(probs * oh2, axis=-1, keepdims=True)
    rf = p1 + p2
    cw_ref[...] = jnp.concatenate(
        [jnp.broadcast_to(p1 / rf, (S, 128)), jnp.broadcast_to(p2 / rf, (S, 128))],
        axis=0)
    ohsum = oh1 + oh2
    r0 = lax.broadcasted_iota(jnp.int32, (CB, CB), 0)
    c0 = lax.broadcasted_iota(jnp.int32, (CB, CB), 1)
    tri = (c0 < r0).astype(jnp.float32)
    carry = jnp.zeros((1, E), jnp.float32)
    excl_rows = []
    for b in range(S // CB):
        blk = ohsum[b * CB:(b + 1) * CB, :]
        excl_rows.append(jnp.dot(tri, blk, preferred_element_type=jnp.float32) + carry)
        carry = carry + jnp.sum(blk, axis=0, keepdims=True)
    excl_cum = jnp.concatenate(excl_rows, axis=0)
    counts = carry
    cap = jnp.floor((counts + (BLK - 1)) * (1.0 / BLK)) * BLK
    iu = lax.broadcasted_iota(jnp.int32, (E, E), 0)
    ju = lax.broadcasted_iota(jnp.int32, (E, E), 1)
    upp = (iu < ju).astype(jnp.float32)
    starts = jnp.dot(cap, upp, preferred_element_type=jnp.float32)
    rank1 = jnp.sum(excl_cum * oh1, axis=-1, keepdims=True)
    rank2 = jnp.sum(excl_cum * oh2, axis=-1, keepdims=True)
    s1 = jnp.sum(starts * oh1, axis=-1, keepdims=True)
    s2 = jnp.sum(starts * oh2, axis=-1, keepdims=True)
    pos_ref[...] = jnp.concatenate([s1 + rank1, s2 + rank2], axis=0).astype(jnp.int32)
    ends = starts + cap
    jt = (lax.broadcasted_iota(jnp.int32, (TE_N, E), 0) * BLK).astype(jnp.float32)
    te = jnp.sum((jt >= ends).astype(jnp.int32), axis=-1, keepdims=True)
    te = jnp.minimum(te, E - 1)
    row_id = lax.broadcasted_iota(jnp.int32, (TE_N, 1), 0)
    used = (ends[0, E - 1] * (1.0 / BLK)).astype(jnp.int32)
    te_ref[...] = jnp.where(row_id == TE_N - 1, used, te)


def _ffn_body(te_ref, x_ref, wg_ref, wl_ref, ps_ref, cw_ref, y_ref):
    i = pl.program_id(0)

    @pl.when(i < te_ref[TE_N - 1])
    def _():
        e = te_ref[i]
        xb = x_ref[...]
        wg = wg_ref[0]
        h1 = lax.dot_general(xb, wg[:F], (((1,), (1,)), ((), ())),
                             preferred_element_type=jnp.float32)
        h2 = lax.dot_general(xb, wg[F:], (((1,), (1,)), ((), ())),
                             preferred_element_type=jnp.float32)
        act = jax.nn.gelu(h1) * h2
        y = lax.dot_general(act, wl_ref[0], (((1,), (0,)), ((), ())),
                            preferred_element_type=jnp.float32)
        y_ref[...] = y * (cw_ref[:, 0:1] * ps_ref[e])


NTOK = S // NW


def _dispatch_sc_body(x_hbm, pos_hbm, cwe_hbm, xs_hbm, cws_hbm,
                      xrow_v, p0_v, p1_v, cw0_v, cw1_v, sem):
    wid = lax.axis_index("s") * 2 + lax.axis_index("c")
    tb = wid * NTOK
    l1 = pltpu.async_copy(x_hbm.at[pl.ds(tb, NTOK)], xrow_v, sem)
    l2 = pltpu.async_copy(pos_hbm.at[pl.ds(tb, NTOK)], p0_v, sem)
    l3 = pltpu.async_copy(pos_hbm.at[pl.ds(S + tb, NTOK)], p1_v, sem)
    l4 = pltpu.async_copy(cwe_hbm.at[pl.ds(tb, NTOK)], cw0_v, sem)
    l5 = pltpu.async_copy(cwe_hbm.at[pl.ds(S + tb, NTOK)], cw1_v, sem)
    l1.wait()
    l2.wait()
    l3.wait()
    l4.wait()
    l5.wait()
    c1 = pltpu.async_copy(xrow_v, xs_hbm.at[p0_v], sem)
    c2 = pltpu.async_copy(xrow_v, xs_hbm.at[p1_v], sem)
    c3 = pltpu.async_copy(cw0_v, cws_hbm.at[p0_v], sem)
    c4 = pltpu.async_copy(cw1_v, cws_hbm.at[p1_v], sem)
    c1.wait()
    c2.wait()
    c3.wait()
    c4.wait()


def _collect_sc_body(y_hbm, pos_hbm, o_hbm, p0_v, p1_v, rows_a, rows_b, sem):
    wid = lax.axis_index("s") * 2 + lax.axis_index("c")
    tb = wid * NTOK
    pltpu.sync_copy(pos_hbm.at[pl.ds(tb, NTOK)], p0_v)
    pltpu.sync_copy(pos_hbm.at[pl.ds(S + tb, NTOK)], p1_v)
    g1 = pltpu.async_copy(y_hbm.at[p0_v], rows_a, sem)
    g2 = pltpu.async_copy(y_hbm.at[p1_v], rows_b, sem)
    g1.wait()
    g2.wait()

    @plsc.parallel_loop(0, NTOK)
    def _(t):
        for c in range(D // 16):
            a = rows_a[t, pl.ds(c * 16, 16)]
            b = rows_b[t, pl.ds(c * 16, 16)]
            rows_a[t, pl.ds(c * 16, 16)] = a + b

    pltpu.sync_copy(rows_a, o_hbm.at[pl.ds(tb, NTOK)])


def _sc_mesh():
    return plsc.VectorSubcoreMesh(core_axis_name="c", subcore_axis_name="s")


def kernel(x, router_scale, per_expert_scale, w_router, w_gating, w_linear):
    x2 = x.reshape(S, D)
    cw, pos, te = pl.pallas_call(
        _router_body,
        out_shape=[
            jax.ShapeDtypeStruct((P, 128), jnp.float32),
            jax.ShapeDtypeStruct((P, 1), jnp.int32),
            jax.ShapeDtypeStruct((TE_N, 1), jnp.int32),
        ],
    )(x2, router_scale.reshape(1, D), w_router)
    pos_flat = pos.reshape(P)

    dispatch = pl.kernel(
        _dispatch_sc_body,
        out_type=(
            jax.ShapeDtypeStruct((ROWS, D), jnp.float32),
            jax.ShapeDtypeStruct((ROWS, 128), jnp.float32),
        ),
        mesh=_sc_mesh(),
        scratch_types=[
            pltpu.VMEM((NTOK, D), jnp.float32),
            pltpu.VMEM((NTOK,), jnp.int32),
            pltpu.VMEM((NTOK,), jnp.int32),
            pltpu.VMEM((NTOK, 128), jnp.float32),
            pltpu.VMEM((NTOK, 128), jnp.float32),
            pltpu.SemaphoreType.DMA,
        ],
    )
    xs, cws = dispatch(x2, pos_flat, cw)

    te_arr = te.reshape(TE_N)
    grid_spec = pltpu.PrefetchScalarGridSpec(
        num_scalar_prefetch=1,
        grid=(NT,),
        in_specs=[
            pl.BlockSpec((BLK, D), lambda i, te: (i, 0)),
            pl.BlockSpec((1, 2 * F, D), lambda i, te: (te[i], 0, 0)),
            pl.BlockSpec((1, F, D), lambda i, te: (te[i], 0, 0)),
            pl.BlockSpec(memory_space=pltpu.SMEM),
            pl.BlockSpec((BLK, 128), lambda i, te: (i, 0)),
        ],
        out_specs=pl.BlockSpec((BLK, D), lambda i, te: (i, 0)),
    )
    y = pl.pallas_call(
        _ffn_body,
        grid_spec=grid_spec,
        out_shape=jax.ShapeDtypeStruct((ROWS, D), jnp.float32),
    )(te_arr, xs, w_gating.reshape(E, 2 * F, D), w_linear, per_expert_scale, cws)

    collect = pl.kernel(
        _collect_sc_body,
        out_type=jax.ShapeDtypeStruct((S, D), jnp.float32),
        mesh=_sc_mesh(),
        scratch_types=[
            pltpu.VMEM((NTOK,), jnp.int32),
            pltpu.VMEM((NTOK,), jnp.int32),
            pltpu.VMEM((NTOK, D), jnp.float32),
            pltpu.VMEM((NTOK, D), jnp.float32),
            pltpu.SemaphoreType.DMA,
        ],
    )
    out = collect(y, pos_flat)
    return out.reshape(1, S, D)


# speedup vs baseline: 1.3806x; 1.0900x over previous
import functools

import jax
import jax.numpy as jnp
from jax import lax
from jax.experimental import pallas as pl
from jax.experimental.pallas import tpu as pltpu
from jax.experimental.pallas import tpu_sc as plsc

S, D, E, K, F = 2048, 768, 8, 2, 2048
P = S * K
BLK = 512
NT = (P // BLK) + E - 1
TE_N = 64
ROWS = NT * BLK
NW = 32
CHUNK = P // NW
CB = 512


def _router_body(x_ref, rs_ref, wr_ref, cw_ref, pos_ref, te_ref):
    x = x_ref[...]
    xr = x * lax.rsqrt(jnp.mean(x * x, axis=-1, keepdims=True) + 1e-6)
    xr = xr * (lax.rsqrt(jnp.float32(D)) * rs_ref[...])
    logits = jnp.dot(xr, wr_ref[...], preferred_element_type=jnp.float32)
    m = jnp.max(logits, axis=-1, keepdims=True)
    ex = jnp.exp(logits - m)
    probs = ex / jnp.sum(ex, axis=-1, keepdims=True)
    idx = lax.broadcasted_iota(jnp.int32, (S, E), 1)
    a1 = jnp.min(jnp.where(logits == m, idx, E), axis=-1, keepdims=True)
    l2 = jnp.where(idx == a1, -jnp.inf, logits)
    m2 = jnp.max(l2, axis=-1, keepdims=True)
    a2 = jnp.min(jnp.where(l2 == m2, idx, E), axis=-1, keepdims=True)
    oh1 = (idx == a1).astype(jnp.float32)
    oh2 = (idx == a2).astype(jnp.float32)
    p1 = jnp.sum(probs * oh1, axis=-1, keepdims=True)
    p2 = jnp.sum(probs * oh2, axis=-1, keepdims=True)
    rf = p1 + p2
    cw_ref[...] = jnp.concatenate(
        [jnp.broadcast_to(p1 / rf, (S, 128)), jnp.broadcast_to(p2 / rf, (S, 128))],
        axis=0)
    ohsum = oh1 + oh2
    r0 = lax.broadcasted_iota(jnp.int32, (CB, CB), 0)
    c0 = lax.broadcasted_iota(jnp.int32, (CB, CB), 1)
    tri = (c0 < r0).astype(jnp.float32)
    carry = jnp.zeros((1, E), jnp.float32)
    excl_rows = []
    for b in range(S // CB):
        blk = ohsum[b * CB:(b + 1) * CB, :]
        excl_rows.append(jnp.dot(tri, blk, preferred_element_type=jnp.float32) + carry)
        carry = carry + jnp.sum(blk, axis=0, keepdims=True)
    excl_cum = jnp.concatenate(excl_rows, axis=0)
    counts = carry
    cap = jnp.floor((counts + (BLK - 1)) * (1.0 / BLK)) * BLK
    iu = lax.broadcasted_iota(jnp.int32, (E, E), 0)
    ju = lax.broadcasted_iota(jnp.int32, (E, E), 1)
    upp = (iu < ju).astype(jnp.float32)
    starts = jnp.dot(cap, upp, preferred_element_type=jnp.float32)
    rank1 = jnp.sum(excl_cum * oh1, axis=-1, keepdims=True)
    rank2 = jnp.sum(excl_cum * oh2, axis=-1, keepdims=True)
    s1 = jnp.sum(starts * oh1, axis=-1, keepdims=True)
    s2 = jnp.sum(starts * oh2, axis=-1, keepdims=True)
    pos_ref[...] = jnp.concatenate([s1 + rank1, s2 + rank2], axis=0).astype(jnp.int32)
    ends = starts + cap
    jt = (lax.broadcasted_iota(jnp.int32, (TE_N, E), 0) * BLK).astype(jnp.float32)
    te = jnp.sum((jt >= ends).astype(jnp.int32), axis=-1, keepdims=True)
    te = jnp.minimum(te, E - 1)
    row_id = lax.broadcasted_iota(jnp.int32, (TE_N, 1), 0)
    used = (ends[0, E - 1] * (1.0 / BLK)).astype(jnp.int32)
    te_ref[...] = jnp.where(row_id == TE_N - 1, used, te)


def _ffn_body(te_ref, x_ref, wg_ref, wl_ref, ps_ref, cw_ref, y_ref):
    i = pl.program_id(0)

    @pl.when(i < te_ref[TE_N - 1])
    def _():
        e = te_ref[i]
        xb = x_ref[...]
        wg = wg_ref[0]
        h1 = lax.dot_general(xb, wg[:F], (((1,), (1,)), ((), ())),
                             preferred_element_type=jnp.float32)
        h2 = lax.dot_general(xb, wg[F:], (((1,), (1,)), ((), ())),
                             preferred_element_type=jnp.float32)
        act = jax.nn.gelu(h1) * h2
        y = lax.dot_general(act, wl_ref[0], (((1,), (0,)), ((), ())),
                            preferred_element_type=jnp.float32)
        y_ref[...] = y * (cw_ref[:, 0:1] * ps_ref[e])


NTOK = S // NW


def _dispatch_sc_body(x_hbm, pos_hbm, cwe_hbm, xs_hbm, cws_hbm,
                      xrow_v, p0_v, p1_v, cw0_v, cw1_v, sem):
    wid = lax.axis_index("s") * 2 + lax.axis_index("c")
    tb = wid * NTOK
    l1 = pltpu.async_copy(x_hbm.at[pl.ds(tb, NTOK)], xrow_v, sem)
    l2 = pltpu.async_copy(pos_hbm.at[pl.ds(tb, NTOK)], p0_v, sem)
    l3 = pltpu.async_copy(pos_hbm.at[pl.ds(S + tb, NTOK)], p1_v, sem)
    l4 = pltpu.async_copy(cwe_hbm.at[pl.ds(tb, NTOK)], cw0_v, sem)
    l5 = pltpu.async_copy(cwe_hbm.at[pl.ds(S + tb, NTOK)], cw1_v, sem)
    l1.wait()
    l2.wait()
    l3.wait()
    l4.wait()
    l5.wait()
    c1 = pltpu.async_copy(xrow_v, xs_hbm.at[p0_v], sem)
    c2 = pltpu.async_copy(xrow_v, xs_hbm.at[p1_v], sem)
    c3 = pltpu.async_copy(cw0_v, cws_hbm.at[p0_v], sem)
    c4 = pltpu.async_copy(cw1_v, cws_hbm.at[p1_v], sem)
    c1.wait()
    c2.wait()
    c3.wait()
    c4.wait()


def _collect_sc_body(y_hbm, pos_hbm, o_hbm, p0_v, p1_v, rows_a, rows_b, sem):
    wid = lax.axis_index("s") * 2 + lax.axis_index("c")
    tb = wid * NTOK
    pltpu.sync_copy(pos_hbm.at[pl.ds(tb, NTOK)], p0_v)
    pltpu.sync_copy(pos_hbm.at[pl.ds(S + tb, NTOK)], p1_v)
    g1 = pltpu.async_copy(y_hbm.at[p0_v], rows_a, sem)
    g2 = pltpu.async_copy(y_hbm.at[p1_v], rows_b, sem)
    g1.wait()
    g2.wait()

    @plsc.parallel_loop(0, NTOK)
    def _(t):
        for c in range(D // 16):
            a = rows_a[t, pl.ds(c * 16, 16)]
            b = rows_b[t, pl.ds(c * 16, 16)]
            rows_a[t, pl.ds(c * 16, 16)] = a + b

    pltpu.sync_copy(rows_a, o_hbm.at[pl.ds(tb, NTOK)])


def _sc_mesh():
    return plsc.VectorSubcoreMesh(core_axis_name="c", subcore_axis_name="s")


def kernel(x, router_scale, per_expert_scale, w_router, w_gating, w_linear):
    x2 = x.reshape(S, D)
    cw, pos, te = pl.pallas_call(
        _router_body,
        out_shape=[
            jax.ShapeDtypeStruct((P, 128), jnp.float32),
            jax.ShapeDtypeStruct((P, 1), jnp.int32),
            jax.ShapeDtypeStruct((TE_N, 1), jnp.int32),
        ],
    )(x2, router_scale.reshape(1, D), w_router)
    pos_flat = pos.reshape(P)

    dispatch = pl.kernel(
        _dispatch_sc_body,
        out_type=(
            jax.ShapeDtypeStruct((ROWS, D), jnp.float32),
            jax.ShapeDtypeStruct((ROWS, 128), jnp.float32),
        ),
        mesh=_sc_mesh(),
        scratch_types=[
            pltpu.VMEM((NTOK, D), jnp.float32),
            pltpu.VMEM((NTOK,), jnp.int32),
            pltpu.VMEM((NTOK,), jnp.int32),
            pltpu.VMEM((NTOK, 128), jnp.float32),
            pltpu.VMEM((NTOK, 128), jnp.float32),
            pltpu.SemaphoreType.DMA,
        ],
    )
    xs, cws = dispatch(x2, pos_flat, cw)

    te_arr = te.reshape(TE_N)
    grid_spec = pltpu.PrefetchScalarGridSpec(
        num_scalar_prefetch=1,
        grid=(NT,),
        in_specs=[
            pl.BlockSpec((BLK, D), lambda i, te: (i, 0)),
            pl.BlockSpec((1, 2 * F, D), lambda i, te: (te[i], 0, 0)),
            pl.BlockSpec((1, F, D), lambda i, te: (te[i], 0, 0)),
            pl.BlockSpec(memory_space=pltpu.SMEM),
            pl.BlockSpec((BLK, 128), lambda i, te: (i, 0)),
        ],
        out_specs=pl.BlockSpec((BLK, D), lambda i, te: (i, 0)),
    )
    y = pl.pallas_call(
        _ffn_body,
        grid_spec=grid_spec,
        out_shape=jax.ShapeDtypeStruct((ROWS, D), jnp.float32),
    )(te_arr, xs, w_gating.reshape(E, 2 * F, D), w_linear, per_expert_scale, cws)

    collect = pl.kernel(
        _collect_sc_body,
        out_type=jax.ShapeDtypeStruct((S, D), jnp.float32),
        mesh=_sc_mesh(),
        scratch_types=[
            pltpu.VMEM((NTOK,), jnp.int32),
            pltpu.VMEM((NTOK,), jnp.int32),
            pltpu.VMEM((NTOK, D), jnp.float32),
            pltpu.VMEM((NTOK, D), jnp.float32),
            pltpu.SemaphoreType.DMA,
        ],
    )
    out = collect(y, pos_flat)
    return out.reshape(1, S, D)


# speedup vs baseline: 1.3833x; 1.0020x over previous
import jax
import jax.numpy as jnp
from jax import lax
from jax.experimental import pallas as pl
from jax.experimental.pallas import tpu as pltpu
from jax.experimental.pallas import tpu_sc as plsc

S, D, E, K, F = 2048, 768, 8, 2, 2048
P = S * K
BLK = 512
NT = (P // BLK) + E - 1
TE_N = 64
ROWS = NT * BLK
NW = 32
CHUNK = P // NW
CB = 512


def _router_body(x_ref, rs_ref, wr_ref, cw_ref, pos_ref, te_ref):
    x = x_ref[...]
    xr = x * lax.rsqrt(jnp.mean(x * x, axis=-1, keepdims=True) + 1e-6)
    xr = xr * (lax.rsqrt(jnp.float32(D)) * rs_ref[...])
    logits = jnp.dot(xr, wr_ref[...], preferred_element_type=jnp.float32)
    m = jnp.max(logits, axis=-1, keepdims=True)
    ex = jnp.exp(logits - m)
    probs = ex / jnp.sum(ex, axis=-1, keepdims=True)
    idx = lax.broadcasted_iota(jnp.int32, (S, E), 1)
    a1 = jnp.min(jnp.where(logits == m, idx, E), axis=-1, keepdims=True)
    l2 = jnp.where(idx == a1, -jnp.inf, logits)
    m2 = jnp.max(l2, axis=-1, keepdims=True)
    a2 = jnp.min(jnp.where(l2 == m2, idx, E), axis=-1, keepdims=True)
    oh1 = (idx == a1).astype(jnp.float32)
    oh2 = (idx == a2).astype(jnp.float32)
    p1 = jnp.sum(probs * oh1, axis=-1, keepdims=True)
    p2 = jnp.sum(probs * oh2, axis=-1, keepdims=True)
    rf = p1 + p2
    cw_ref[...] = jnp.concatenate(
        [jnp.broadcast_to(p1 / rf, (S, 128)), jnp.broadcast_to(p2 / rf, (S, 128))],
        axis=0)
    ohsum = oh1 + oh2
    r0 = lax.broadcasted_iota(jnp.int32, (CB, CB), 0)
    c0 = lax.broadcasted_iota(jnp.int32, (CB, CB), 1)
    tri = (c0 < r0).astype(jnp.float32)
    carry = jnp.zeros((1, E), jnp.float32)
    excl_rows = []
    for b in range(S // CB):
        blk = ohsum[b * CB:(b + 1) * CB, :]
        excl_rows.append(jnp.dot(tri, blk, preferred_element_type=jnp.float32) + carry)
        carry = carry + jnp.sum(blk, axis=0, keepdims=True)
    excl_cum = jnp.concatenate(excl_rows, axis=0)
    counts = carry
    cap = jnp.floor((counts + (BLK - 1)) * (1.0 / BLK)) * BLK
    iu = lax.broadcasted_iota(jnp.int32, (E, E), 0)
    ju = lax.broadcasted_iota(jnp.int32, (E, E), 1)
    upp = (iu < ju).astype(jnp.float32)
    starts = jnp.dot(cap, upp, preferred_element_type=jnp.float32)
    rank1 = jnp.sum(excl_cum * oh1, axis=-1, keepdims=True)
    rank2 = jnp.sum(excl_cum * oh2, axis=-1, keepdims=True)
    s1 = jnp.sum(starts * oh1, axis=-1, keepdims=True)
    s2 = jnp.sum(starts * oh2, axis=-1, keepdims=True)
    pos_ref[...] = jnp.concatenate([s1 + rank1, s2 + rank2], axis=0).astype(jnp.int32)
    ends = starts + cap
    jt = (lax.broadcasted_iota(jnp.int32, (TE_N, E), 0) * BLK).astype(jnp.float32)
    te = jnp.sum((jt >= ends).astype(jnp.int32), axis=-1, keepdims=True)
    te = jnp.minimum(te, E - 1)
    row_id = lax.broadcasted_iota(jnp.int32, (TE_N, 1), 0)
    used = (ends[0, E - 1] * (1.0 / BLK)).astype(jnp.int32)
    te_ref[...] = jnp.where(row_id == TE_N - 1, used, te)


def _ffn_body(te_ref, x_ref, wg_ref, wl_ref, ps_ref, cw_ref, y_ref):
    i = pl.program_id(0)

    @pl.when(i < te_ref[TE_N - 1])
    def _():
        e = te_ref[i]
        xb = x_ref[...]
        wg = wg_ref[0]
        h1 = lax.dot_general(xb, wg[:F], (((1,), (1,)), ((), ())),
                             preferred_element_type=jnp.float32)
        h2 = lax.dot_general(xb, wg[F:], (((1,), (1,)), ((), ())),
                             preferred_element_type=jnp.float32)
        act = jax.nn.gelu(h1) * h2
        y = lax.dot_general(act, wl_ref[0], (((1,), (0,)), ((), ())),
                            preferred_element_type=jnp.float32)
        y_ref[...] = y * (cw_ref[:, 0:1] * ps_ref[e])


NTOK = S // NW


def _dispatch_sc_body(x_hbm, pos_hbm, cwe_hbm, xs_hbm, cws_hbm,
                      xrow_v, p0_v, p1_v, cw0_v, cw1_v, sem):
    wid = lax.axis_index("s") * 2 + lax.axis_index("c")
    tb = wid * NTOK
    l1 = pltpu.async_copy(x_hbm.at[pl.ds(tb, NTOK)], xrow_v, sem)
    l2 = pltpu.async_copy(pos_hbm.at[pl.ds(tb, NTOK)], p0_v, sem)
    l3 = pltpu.async_copy(pos_hbm.at[pl.ds(S + tb, NTOK)], p1_v, sem)
    l4 = pltpu.async_copy(cwe_hbm.at[pl.ds(tb, NTOK)], cw0_v, sem)
    l5 = pltpu.async_copy(cwe_hbm.at[pl.ds(S + tb, NTOK)], cw1_v, sem)
    l1.wait()
    l2.wait()
    l3.wait()
    l4.wait()
    l5.wait()
    c1 = pltpu.async_copy(xrow_v, xs_hbm.at[p0_v], sem)
    c2 = pltpu.async_copy(xrow_v, xs_hbm.at[p1_v], sem)
    c3 = pltpu.async_copy(cw0_v, cws_hbm.at[p0_v], sem)
    c4 = pltpu.async_copy(cw1_v, cws_hbm.at[p1_v], sem)
    c1.wait()
    c2.wait()
    c3.wait()
    c4.wait()


def _collect_sc_body(y_hbm, pos_hbm, o_hbm, p0_v, p1_v, rows_a, rows_b, sem):
    wid = lax.axis_index("s") * 2 + lax.axis_index("c")
    tb = wid * NTOK
    pltpu.sync_copy(pos_hbm.at[pl.ds(tb, NTOK)], p0_v)
    pltpu.sync_copy(pos_hbm.at[pl.ds(S + tb, NTOK)], p1_v)
    g1 = pltpu.async_copy(y_hbm.at[p0_v], rows_a, sem)
    g2 = pltpu.async_copy(y_hbm.at[p1_v], rows_b, sem)
    g1.wait()
    g2.wait()

    @plsc.parallel_loop(0, NTOK)
    def _(t):
        for c in range(D // 16):
            a = rows_a[t, pl.ds(c * 16, 16)]
            b = rows_b[t, pl.ds(c * 16, 16)]
            rows_a[t, pl.ds(c * 16, 16)] = a + b

    pltpu.sync_copy(rows_a, o_hbm.at[pl.ds(tb, NTOK)])


def _sc_mesh():
    return plsc.VectorSubcoreMesh(core_axis_name="c", subcore_axis_name="s")


def kernel(x, router_scale, per_expert_scale, w_router, w_gating, w_linear):
    x2 = x.reshape(S, D)
    cw, pos, te = pl.pallas_call(
        _router_body,
        out_shape=[
            jax.ShapeDtypeStruct((P, 128), jnp.float32),
            jax.ShapeDtypeStruct((P, 1), jnp.int32),
            jax.ShapeDtypeStruct((TE_N, 1), jnp.int32),
        ],
    )(x2, router_scale.reshape(1, D), w_router)
    pos_flat = pos.reshape(P)

    dispatch = pl.kernel(
        _dispatch_sc_body,
        out_type=(
            jax.ShapeDtypeStruct((ROWS, D), jnp.float32),
            jax.ShapeDtypeStruct((ROWS, 128), jnp.float32),
        ),
        mesh=_sc_mesh(),
        scratch_types=[
            pltpu.VMEM((NTOK, D), jnp.float32),
            pltpu.VMEM((NTOK,), jnp.int32),
            pltpu.VMEM((NTOK,), jnp.int32),
            pltpu.VMEM((NTOK, 128), jnp.float32),
            pltpu.VMEM((NTOK, 128), jnp.float32),
            pltpu.SemaphoreType.DMA,
        ],
    )
    xs, cws = dispatch(x2, pos_flat, cw)

    te_arr = te.reshape(TE_N)
    grid_spec = pltpu.PrefetchScalarGridSpec(
        num_scalar_prefetch=1,
        grid=(NT,),
        in_specs=[
            pl.BlockSpec((BLK, D), lambda i, te: (i, 0)),
            pl.BlockSpec((1, 2 * F, D), lambda i, te: (te[i], 0, 0)),
            pl.BlockSpec((1, F, D), lambda i, te: (te[i], 0, 0)),
            pl.BlockSpec(memory_space=pltpu.SMEM),
            pl.BlockSpec((BLK, 128), lambda i, te: (i, 0)),
        ],
        out_specs=pl.BlockSpec((BLK, D), lambda i, te: (i, 0)),
    )
    y = pl.pallas_call(
        _ffn_body,
        grid_spec=grid_spec,
        out_shape=jax.ShapeDtypeStruct((ROWS, D), jnp.float32),
    )(te_arr, xs, w_gating.reshape(E, 2 * F, D), w_linear, per_expert_scale, cws)

    collect = pl.kernel(
        _collect_sc_body,
        out_type=jax.ShapeDtypeStruct((S, D), jnp.float32),
        mesh=_sc_mesh(),
        scratch_types=[
            pltpu.VMEM((NTOK,), jnp.int32),
            pltpu.VMEM((NTOK,), jnp.int32),
            pltpu.VMEM((NTOK, D), jnp.float32),
            pltpu.VMEM((NTOK, D), jnp.float32),
            pltpu.SemaphoreType.DMA,
        ],
    )
    out = collect(y, pos_flat)
    return out.reshape(1, S, D)
